# baseline (speedup 1.0000x reference)
import functools

import jax
import jax.numpy as jnp
import numpy as np
from jax import lax
from jax.experimental import pallas as pl
from jax.experimental.pallas import tpu as pltpu
from jax.experimental.pallas import tpu_sc as plsc

N = 10000
E = 320000
D = 128
H = 8
C = 16
G = 128
OUT = 16

NC = 2
NS = 16
NW = NC * NS
NP = 10240
EP = 327680
EPW = EP // NW
K = 16
NCHUNK = EPW // K
NBUF = 5
NCH0 = 480
NCH1 = 2 * NCHUNK - NCH0
ROW = 144
ADW = 16
RPT = NP // NS
BLK = 512
NBLK = NP // BLK


_PERM = np.array([(j % 8) * 16 + 2 * (j // 16) + ((j % 16) // 8)
                  for j in range(D)], dtype=np.int32)


def _head_sel(dtype):
    jj = lax.broadcasted_iota(jnp.int32, (D, H), 0)
    hh = lax.broadcasted_iota(jnp.int32, (D, H), 1)
    return jnp.where((jj & 7) == hh, 1.0, 0.0).astype(dtype)


def _tc_prep_body(x_ref, w_ref, asv_ref, adv_ref, xw_ref, as_ref, ad_ref):
    xw = jnp.dot(x_ref[...], w_ref[...], preferred_element_type=jnp.float32)
    sel = _head_sel(jnp.float32)
    as_ref[...] = jnp.dot(xw * asv_ref[...], sel, preferred_element_type=jnp.float32)
    ad_ref[...] = jnp.dot(xw * adv_ref[...], sel, preferred_element_type=jnp.float32)
    xw_ref[...] = xw


def _tc_prep(xp, W, asv, adv):
    return pl.pallas_call(
        _tc_prep_body,
        grid=(NBLK,),
        in_specs=[
            pl.BlockSpec((BLK, D), lambda i: (i, 0)),
            pl.BlockSpec((D, D), lambda i: (0, 0)),
            pl.BlockSpec((1, D), lambda i: (0, 0)),
            pl.BlockSpec((1, D), lambda i: (0, 0)),
        ],
        out_specs=[
            pl.BlockSpec((BLK, D), lambda i: (i, 0)),
            pl.BlockSpec((BLK, H), lambda i: (i, 0)),
            pl.BlockSpec((BLK, H), lambda i: (i, 0)),
        ],
        out_shape=[
            jax.ShapeDtypeStruct((NP, D), jnp.float32),
            jax.ShapeDtypeStruct((NP, H), jnp.float32),
            jax.ShapeDtypeStruct((NP, H), jnp.float32),
        ],
    )(xp, W, asv, adv)


def _sc_edge_body(packed_hbm, adst_hbm, src_hbm, dst_hbm, out_hbm,
                  srcs_v, dsts_v, rows_v, adst_v, acc_sh,
                  sems_g, sems_s):
    c = lax.axis_index("c")
    s = lax.axis_index("s")
    nch = jnp.where(c == 0, NCH0, NCH1)
    base = jnp.where(c == 0, s * NCH0, NS * NCH0 + s * NCH1)

    @pl.when(c == 0)
    def _():
        pltpu.sync_copy(src_hbm.at[pl.ds(base, NCH0)], srcs_v.at[pl.ds(0, NCH0)])
        pltpu.sync_copy(dst_hbm.at[pl.ds(base, NCH0)], dsts_v.at[pl.ds(0, NCH0)])

    @pl.when(c == 1)
    def _():
        pltpu.sync_copy(src_hbm.at[pl.ds(base, NCH1)], srcs_v.at[pl.ds(0, NCH1)])
        pltpu.sync_copy(dst_hbm.at[pl.ds(base, NCH1)], dsts_v.at[pl.ds(0, NCH1)])

    def zero_row(r, _):
        for j in range(ROW // 16):
            rows_v[0, r, pl.ds(j * 16, 16)] = jnp.zeros((16,), jnp.float32)
        return 0
    lax.fori_loop(0, K, zero_row, 0)
    for kk in range(RPT // K):
        pltpu.make_async_copy(rows_v.at[0],
                              acc_sh.at[pl.ds(s * RPT + kk * K, K)],
                              sems_g.at[0]).start()
    for kk in range(RPT // K):
        pltpu.make_async_copy(rows_v.at[0],
                              acc_sh.at[pl.ds(s * RPT + kk * K, K)],
                              sems_g.at[0]).wait()
    plsc.subcore_barrier()

    lane = lax.iota(jnp.int32, 16)

    def start_g(ci, b):
        pltpu.make_async_copy(packed_hbm.at[srcs_v.at[ci]], rows_v.at[b],
                              sems_g.at[b]).start()
        pltpu.make_async_copy(adst_hbm.at[dsts_v.at[ci]], adst_v.at[b],
                              sems_g.at[b]).start()

    def wait_g(ci, b):
        pltpu.make_async_copy(packed_hbm.at[srcs_v.at[ci]], rows_v.at[b],
                              sems_g.at[b]).wait()
        pltpu.make_async_copy(adst_hbm.at[dsts_v.at[ci]], adst_v.at[b],
                              sems_g.at[b]).wait()

    def start_s(ci, b):
        pltpu.make_async_copy(rows_v.at[b], acc_sh.at[dsts_v.at[ci]],
                              sems_s.at[b]).start(add=True)

    def wait_s(ci, b):
        pltpu.make_async_copy(rows_v.at[b], acc_sh.at[dsts_v.at[ci]],
                              sems_s.at[b]).wait()

    idx8 = lane & 7

    def compute(b):
        for i in range(K):
            a = rows_v[b, i, pl.ds(D, 16)] + adst_v[b, i, :]
            a = jnp.where(a < 0, a * 0.2, a)
            sv = jnp.exp(a)
            sv = jnp.where(lane < H, sv, 0.0)
            rows_v[b, i, pl.ds(D, 16)] = sv
            sp = sv.at[idx8].get(mode="promise_in_bounds")
            for h in range(H):
                rows_v[b, i, pl.ds(h * 16, 16)] = (
                    rows_v[b, i, pl.ds(h * 16, 16)] * sp)

    start_g(0, 0)
    start_g(1, 1)

    def ring_body(p, _):
        for j in range(NBUF):
            ci = NBUF * p + j
            b2 = (j + 2) % NBUF
            wait_g(ci, j)

            @pl.when(jnp.logical_and(ci >= NBUF - 2, ci + 2 < nch))
            def _():
                wait_s(ci - (NBUF - 2), b2)

            @pl.when(ci + 2 < nch)
            def _():
                start_g(ci + 2, b2)
            compute(j)
            start_s(ci, j)
        return 0
    lax.fori_loop(0, nch // NBUF, ring_body, 0)
    for j in range(NBUF):
        wait_s(nch - NBUF + j, j)

    plsc.subcore_barrier()
    pltpu.sync_copy(acc_sh.at[pl.ds(s * RPT, RPT)],
                    out_hbm.at[c, pl.ds(s * RPT, RPT)])


def _sc_edge(packed, adst, srcp, dstp):
    return pl.kernel(
        _sc_edge_body,
        out_type=jax.ShapeDtypeStruct((NC, NP, ROW), jnp.float32),
        mesh=plsc.VectorSubcoreMesh(core_axis_name="c", subcore_axis_name="s",
                                    num_cores=NC, num_subcores=NS),
        compiler_params=pltpu.CompilerParams(use_tc_tiling_on_sc=False),
        scratch_types=[
            pltpu.VMEM((NCH1, K), jnp.int32),
            pltpu.VMEM((NCH1, K), jnp.int32),
            pltpu.VMEM((NBUF, K, ROW), jnp.float32),
            pltpu.VMEM((NBUF, K, ADW), jnp.float32),
            pltpu.VMEM_SHARED((NP, ROW), jnp.float32),
            pltpu.SemaphoreType.DMA((NBUF,)),
            pltpu.SemaphoreType.DMA((NBUF,)),
        ],
    )(packed, adst, srcp, dstp)


def _denom_sel():
    hh = lax.broadcasted_iota(jnp.int32, (H, D), 0)
    jj = lax.broadcasted_iota(jnp.int32, (H, D), 1)
    return jnp.where((jj & 7) == hh, 1.0, 0.0)


def _combine_norm(acc_ref, b_ref, blk_idx):
    a = acc_ref[0] + acc_ref[1]
    msg = a[:, :D]
    dn = a[:, D:D + H]
    d128 = jnp.dot(dn, _denom_sel(), preferred_element_type=jnp.float32)
    hv = msg / (d128 + 1e-16) + b_ref[...]
    hv = jnp.where(hv > 0, hv, jnp.exp(hv) - 1.0)
    rows = blk_idx * BLK + lax.broadcasted_iota(jnp.int32, (BLK, 1), 0)
    return jnp.where(rows < N, hv, 0.0)


def _tc_mid_body(acc_ref, b_ref, w_ref, asv_ref, adv_ref, xw_ref, as_ref, ad_ref):
    i = pl.program_id(0)
    h1 = _combine_norm(acc_ref, b_ref, i)
    xw = jnp.dot(h1, w_ref[...], preferred_element_type=jnp.float32)
    sel = _head_sel(jnp.float32)
    as_ref[...] = jnp.dot(xw * asv_ref[...], sel, preferred_element_type=jnp.float32)
    ad_ref[...] = jnp.dot(xw * adv_ref[...], sel, preferred_element_type=jnp.float32)
    xw_ref[...] = xw


def _tc_mid(acc, bv, W, asv, adv):
    return pl.pallas_call(
        _tc_mid_body,
        grid=(NBLK,),
        in_specs=[
            pl.BlockSpec((NC, BLK, ROW), lambda i: (0, i, 0)),
            pl.BlockSpec((1, D), lambda i: (0, 0)),
            pl.BlockSpec((D, D), lambda i: (0, 0)),
            pl.BlockSpec((1, D), lambda i: (0, 0)),
            pl.BlockSpec((1, D), lambda i: (0, 0)),
        ],
        out_specs=[
            pl.BlockSpec((BLK, D), lambda i: (i, 0)),
            pl.BlockSpec((BLK, H), lambda i: (i, 0)),
            pl.BlockSpec((BLK, H), lambda i: (i, 0)),
        ],
        out_shape=[
            jax.ShapeDtypeStruct((NP, D), jnp.float32),
            jax.ShapeDtypeStruct((NP, H), jnp.float32),
            jax.ShapeDtypeStruct((NP, H), jnp.float32),
        ],
    )(acc, bv, W, asv, adv)


def _tc_final_body(acc_ref, b_ref, batch_ref, lw_ref, lb_ref, out_ref, pool_ref):
    i = pl.program_id(0)

    @pl.when(i == 0)
    def _():
        pool_ref[...] = jnp.zeros((G, ROW), jnp.float32)

    h2 = _combine_norm(acc_ref, b_ref, i)
    h2e = jnp.concatenate([h2, jnp.ones((BLK, ROW - D), jnp.float32)], axis=1)
    bv = batch_ref[0]
    gg = lax.broadcasted_iota(jnp.int32, (G, BLK), 0).astype(jnp.float32)
    p = jnp.where(gg == bv, 1.0, 0.0)
    pool_ref[...] += jnp.dot(p, h2e, preferred_element_type=jnp.float32)

    @pl.when(i == NBLK - 1)
    def _():
        sums = pool_ref[:, :D]
        counts = pool_ref[:, D:D + 1]
        pooled = sums / jnp.maximum(counts, 1.0)
        out_ref[...] = jnp.dot(pooled, lw_ref[...],
                               preferred_element_type=jnp.float32) + lb_ref[...]


def _tc_final(acc, bv, batch2d, lw, lb):
    return pl.pallas_call(
        _tc_final_body,
        grid=(NBLK,),
        in_specs=[
            pl.BlockSpec((NC, BLK, ROW), lambda i: (0, i, 0)),
            pl.BlockSpec((1, D), lambda i: (0, 0)),
            pl.BlockSpec((1, 1, BLK), lambda i: (i, 0, 0)),
            pl.BlockSpec((D, OUT), lambda i: (0, 0)),
            pl.BlockSpec((1, OUT), lambda i: (0, 0)),
        ],
        out_specs=pl.BlockSpec((G, OUT), lambda i: (0, 0)),
        out_shape=jax.ShapeDtypeStruct((G, OUT), jnp.float32),
        scratch_shapes=[pltpu.VMEM((G, ROW), jnp.float32)],
    )(acc, bv, batch2d, lw, lb)


def kernel(x, edge_index, batch, W1, att_src1, att_dst1, b1,
           W2, att_src2, att_dst2, b2, lin_W, lin_b):
    f32 = jnp.float32
    xp = jnp.pad(x, ((0, NP - N), (0, 0)))
    srcp = jnp.concatenate([edge_index[0], jnp.full((EP - E,), N, jnp.int32)]).reshape(EP // K, K)
    dstp = jnp.concatenate([edge_index[1], jnp.full((EP - E,), N, jnp.int32)]).reshape(EP // K, K)
    batch2d = jnp.pad(batch, (0, NP - N), constant_values=G).astype(f32).reshape(NBLK, 1, BLK)

    pm = jnp.asarray(_PERM)
    as1 = att_src1.reshape(D)[pm].reshape(1, D)
    ad1 = att_dst1.reshape(D)[pm].reshape(1, D)
    as2 = att_src2.reshape(D)[pm].reshape(1, D)
    ad2 = att_dst2.reshape(D)[pm].reshape(1, D)
    b1v = b1[pm].reshape(1, D)
    b2v = b2[pm].reshape(1, D)
    lbv = lin_b.reshape(1, OUT)
    W1p = W1[:, pm]
    W2p = W2[pm][:, pm]
    lin_Wp = lin_W[pm]

    xw1, asrc1, adst1 = _tc_prep(xp, W1p, as1, ad1)
    packed1 = jnp.concatenate([xw1, asrc1, adst1], axis=1)
    adst1t = jnp.concatenate([adst1, jnp.zeros((NP, ADW - H), f32)], axis=1)
    acc1 = _sc_edge(packed1, adst1t, srcp, dstp)

    xw2, asrc2, adst2 = _tc_mid(acc1, b1v, W2p, as2, ad2)
    packed2 = jnp.concatenate([xw2, asrc2, adst2], axis=1)
    adst2t = jnp.concatenate([adst2, jnp.zeros((NP, ADW - H), f32)], axis=1)
    acc2 = _sc_edge(packed2, adst2t, srcp, dstp)

    return _tc_final(acc2, b2v, batch2d, lin_Wp, lbv)


# speedup vs baseline: 1.1981x; 1.1981x over previous
import functools

import jax
import jax.numpy as jnp
import numpy as np
from jax import lax
from jax.experimental import pallas as pl
from jax.experimental.pallas import tpu as pltpu
from jax.experimental.pallas import tpu_sc as plsc

N = 10000
E = 320000
D = 128
H = 8
C = 16
G = 128
OUT = 16

NC = 2
NS = 16
NW = NC * NS
NP = 10240
EP = 327680
EPW = EP // NW
K = 16
NCHUNK = EPW // K
NBUF = 5
NCH0 = 800
NCH1 = 2 * NCHUNK - NCH0
ROW = 144
ADW = 16
RPT = NP // NS
BLK = 512
NBLK = NP // BLK


_PERM = np.array([(j % 8) * 16 + 2 * (j // 16) + ((j % 16) // 8)
                  for j in range(D)], dtype=np.int32)


def _head_sel(dtype):
    jj = lax.broadcasted_iota(jnp.int32, (D, H), 0)
    hh = lax.broadcasted_iota(jnp.int32, (D, H), 1)
    return jnp.where((jj & 7) == hh, 1.0, 0.0).astype(dtype)


def _tc_prep_body(x_ref, w_ref, asv_ref, adv_ref, xw_ref, as_ref, ad_ref):
    xw = jnp.dot(x_ref[...], w_ref[...], preferred_element_type=jnp.float32)
    sel = _head_sel(jnp.float32)
    as_ref[...] = jnp.dot(xw * asv_ref[...], sel, preferred_element_type=jnp.float32)
    ad_ref[...] = jnp.dot(xw * adv_ref[...], sel, preferred_element_type=jnp.float32)
    xw_ref[...] = xw


def _tc_prep(xp, W, asv, adv):
    return pl.pallas_call(
        _tc_prep_body,
        grid=(NBLK,),
        in_specs=[
            pl.BlockSpec((BLK, D), lambda i: (i, 0)),
            pl.BlockSpec((D, D), lambda i: (0, 0)),
            pl.BlockSpec((1, D), lambda i: (0, 0)),
            pl.BlockSpec((1, D), lambda i: (0, 0)),
        ],
        out_specs=[
            pl.BlockSpec((BLK, D), lambda i: (i, 0)),
            pl.BlockSpec((BLK, H), lambda i: (i, 0)),
            pl.BlockSpec((BLK, H), lambda i: (i, 0)),
        ],
        out_shape=[
            jax.ShapeDtypeStruct((NP, D), jnp.float32),
            jax.ShapeDtypeStruct((NP, H), jnp.float32),
            jax.ShapeDtypeStruct((NP, H), jnp.float32),
        ],
    )(xp, W, asv, adv)


def _sc_edge_body(packed_hbm, adst_hbm, src_hbm, dst_hbm, out_hbm,
                  srcs_v, dsts_v, rows_v, adst_v, acc_sh,
                  sems_g, sems_s):
    c = lax.axis_index("c")
    s = lax.axis_index("s")
    nch = jnp.where(c == 0, NCH0, NCH1)
    base = jnp.where(c == 0, s * NCH0, NS * NCH0 + s * NCH1)

    @pl.when(c == 0)
    def _():
        pltpu.sync_copy(src_hbm.at[pl.ds(base, NCH0)], srcs_v.at[pl.ds(0, NCH0)])
        pltpu.sync_copy(dst_hbm.at[pl.ds(base, NCH0)], dsts_v.at[pl.ds(0, NCH0)])

    @pl.when(c == 1)
    def _():
        pltpu.sync_copy(src_hbm.at[pl.ds(base, NCH1)], srcs_v.at[pl.ds(0, NCH1)])
        pltpu.sync_copy(dst_hbm.at[pl.ds(base, NCH1)], dsts_v.at[pl.ds(0, NCH1)])

    def zero_row(r, _):
        for j in range(ROW // 16):
            rows_v[0, r, pl.ds(j * 16, 16)] = jnp.zeros((16,), jnp.float32)
        return 0
    lax.fori_loop(0, K, zero_row, 0)
    for kk in range(RPT // K):
        pltpu.make_async_copy(rows_v.at[0],
                              acc_sh.at[pl.ds(s * RPT + kk * K, K)],
                              sems_g.at[0]).start()
    for kk in range(RPT // K):
        pltpu.make_async_copy(rows_v.at[0],
                              acc_sh.at[pl.ds(s * RPT + kk * K, K)],
                              sems_g.at[0]).wait()
    plsc.subcore_barrier()

    lane = lax.iota(jnp.int32, 16)

    def start_g(ci, b):
        pltpu.make_async_copy(packed_hbm.at[srcs_v.at[ci]], rows_v.at[b],
                              sems_g.at[b]).start()
        pltpu.make_async_copy(adst_hbm.at[dsts_v.at[ci]], adst_v.at[b],
                              sems_g.at[b]).start()

    def wait_g(ci, b):
        pltpu.make_async_copy(packed_hbm.at[srcs_v.at[ci]], rows_v.at[b],
                              sems_g.at[b]).wait()
        pltpu.make_async_copy(adst_hbm.at[dsts_v.at[ci]], adst_v.at[b],
                              sems_g.at[b]).wait()

    def start_s(ci, b):
        pltpu.make_async_copy(rows_v.at[b], acc_sh.at[dsts_v.at[ci]],
                              sems_s.at[b]).start(add=True)

    def wait_s(ci, b):
        pltpu.make_async_copy(rows_v.at[b], acc_sh.at[dsts_v.at[ci]],
                              sems_s.at[b]).wait()

    idx8 = lane & 7

    def compute(b):
        for i in range(K):
            a = rows_v[b, i, pl.ds(D, 16)] + adst_v[b, i, :]
            a = jnp.where(a < 0, a * 0.2, a)
            sv = jnp.exp(a)
            sv = jnp.where(lane < H, sv, 0.0)
            rows_v[b, i, pl.ds(D, 16)] = sv
            sp = sv.at[idx8].get(mode="promise_in_bounds")
            for h in range(H):
                rows_v[b, i, pl.ds(h * 16, 16)] = (
                    rows_v[b, i, pl.ds(h * 16, 16)] * sp)

    start_g(0, 0)
    start_g(1, 1)

    def ring_body(p, _):
        for j in range(NBUF):
            ci = NBUF * p + j
            b2 = (j + 2) % NBUF
            wait_g(ci, j)

            @pl.when(jnp.logical_and(ci >= NBUF - 2, ci + 2 < nch))
            def _():
                wait_s(ci - (NBUF - 2), b2)

            @pl.when(ci + 2 < nch)
            def _():
                start_g(ci + 2, b2)
            compute(j)
            start_s(ci, j)
        return 0
    lax.fori_loop(0, nch // NBUF, ring_body, 0)
    for j in range(NBUF):
        wait_s(nch - NBUF + j, j)

    plsc.subcore_barrier()
    pltpu.sync_copy(acc_sh.at[pl.ds(s * RPT, RPT)],
                    out_hbm.at[c, pl.ds(s * RPT, RPT)])


def _sc_edge(packed, adst, srcp, dstp):
    return pl.kernel(
        _sc_edge_body,
        out_type=jax.ShapeDtypeStruct((NC, NP, ROW), jnp.float32),
        mesh=plsc.VectorSubcoreMesh(core_axis_name="c", subcore_axis_name="s",
                                    num_cores=NC, num_subcores=NS),
        compiler_params=pltpu.CompilerParams(use_tc_tiling_on_sc=False),
        scratch_types=[
            pltpu.VMEM((max(NCH0, NCH1), K), jnp.int32),
            pltpu.VMEM((max(NCH0, NCH1), K), jnp.int32),
            pltpu.VMEM((NBUF, K, ROW), jnp.float32),
            pltpu.VMEM((NBUF, K, ADW), jnp.float32),
            pltpu.VMEM_SHARED((NP, ROW), jnp.float32),
            pltpu.SemaphoreType.DMA((NBUF,)),
            pltpu.SemaphoreType.DMA((NBUF,)),
        ],
    )(packed, adst, srcp, dstp)


def _denom_sel():
    hh = lax.broadcasted_iota(jnp.int32, (H, D), 0)
    jj = lax.broadcasted_iota(jnp.int32, (H, D), 1)
    return jnp.where((jj & 7) == hh, 1.0, 0.0)


def _combine_norm(acc_ref, b_ref, blk_idx):
    a = acc_ref[0] + acc_ref[1]
    msg = a[:, :D]
    dn = a[:, D:D + H]
    d128 = jnp.dot(dn, _denom_sel(), preferred_element_type=jnp.float32)
    hv = msg / (d128 + 1e-16) + b_ref[...]
    hv = jnp.where(hv > 0, hv, jnp.exp(hv) - 1.0)
    rows = blk_idx * BLK + lax.broadcasted_iota(jnp.int32, (BLK, 1), 0)
    return jnp.where(rows < N, hv, 0.0)


def _tc_mid_body(acc_ref, b_ref, w_ref, asv_ref, adv_ref, xw_ref, as_ref, ad_ref):
    i = pl.program_id(0)
    h1 = _combine_norm(acc_ref, b_ref, i)
    xw = jnp.dot(h1, w_ref[...], preferred_element_type=jnp.float32)
    sel = _head_sel(jnp.float32)
    as_ref[...] = jnp.dot(xw * asv_ref[...], sel, preferred_element_type=jnp.float32)
    ad_ref[...] = jnp.dot(xw * adv_ref[...], sel, preferred_element_type=jnp.float32)
    xw_ref[...] = xw


def _tc_mid(acc, bv, W, asv, adv):
    return pl.pallas_call(
        _tc_mid_body,
        grid=(NBLK,),
        in_specs=[
            pl.BlockSpec((NC, BLK, ROW), lambda i: (0, i, 0)),
            pl.BlockSpec((1, D), lambda i: (0, 0)),
            pl.BlockSpec((D, D), lambda i: (0, 0)),
            pl.BlockSpec((1, D), lambda i: (0, 0)),
            pl.BlockSpec((1, D), lambda i: (0, 0)),
        ],
        out_specs=[
            pl.BlockSpec((BLK, D), lambda i: (i, 0)),
            pl.BlockSpec((BLK, H), lambda i: (i, 0)),
            pl.BlockSpec((BLK, H), lambda i: (i, 0)),
        ],
        out_shape=[
            jax.ShapeDtypeStruct((NP, D), jnp.float32),
            jax.ShapeDtypeStruct((NP, H), jnp.float32),
            jax.ShapeDtypeStruct((NP, H), jnp.float32),
        ],
    )(acc, bv, W, asv, adv)


def _tc_final_body(acc_ref, b_ref, batch_ref, lw_ref, lb_ref, out_ref, pool_ref):
    i = pl.program_id(0)

    @pl.when(i == 0)
    def _():
        pool_ref[...] = jnp.zeros((G, ROW), jnp.float32)

    h2 = _combine_norm(acc_ref, b_ref, i)
    h2e = jnp.concatenate([h2, jnp.ones((BLK, ROW - D), jnp.float32)], axis=1)
    bv = batch_ref[0]
    gg = lax.broadcasted_iota(jnp.int32, (G, BLK), 0).astype(jnp.float32)
    p = jnp.where(gg == bv, 1.0, 0.0)
    pool_ref[...] += jnp.dot(p, h2e, preferred_element_type=jnp.float32)

    @pl.when(i == NBLK - 1)
    def _():
        sums = pool_ref[:, :D]
        counts = pool_ref[:, D:D + 1]
        pooled = sums / jnp.maximum(counts, 1.0)
        out_ref[...] = jnp.dot(pooled, lw_ref[...],
                               preferred_element_type=jnp.float32) + lb_ref[...]


def _tc_final(acc, bv, batch2d, lw, lb):
    return pl.pallas_call(
        _tc_final_body,
        grid=(NBLK,),
        in_specs=[
            pl.BlockSpec((NC, BLK, ROW), lambda i: (0, i, 0)),
            pl.BlockSpec((1, D), lambda i: (0, 0)),
            pl.BlockSpec((1, 1, BLK), lambda i: (i, 0, 0)),
            pl.BlockSpec((D, OUT), lambda i: (0, 0)),
            pl.BlockSpec((1, OUT), lambda i: (0, 0)),
        ],
        out_specs=pl.BlockSpec((G, OUT), lambda i: (0, 0)),
        out_shape=jax.ShapeDtypeStruct((G, OUT), jnp.float32),
        scratch_shapes=[pltpu.VMEM((G, ROW), jnp.float32)],
    )(acc, bv, batch2d, lw, lb)


def kernel(x, edge_index, batch, W1, att_src1, att_dst1, b1,
           W2, att_src2, att_dst2, b2, lin_W, lin_b):
    f32 = jnp.float32
    xp = jnp.pad(x, ((0, NP - N), (0, 0)))
    srcp = jnp.concatenate([edge_index[0], jnp.full((EP - E,), N, jnp.int32)]).reshape(EP // K, K)
    dstp = jnp.concatenate([edge_index[1], jnp.full((EP - E,), N, jnp.int32)]).reshape(EP // K, K)
    batch2d = jnp.pad(batch, (0, NP - N), constant_values=G).astype(f32).reshape(NBLK, 1, BLK)

    pm = jnp.asarray(_PERM)
    as1 = att_src1.reshape(D)[pm].reshape(1, D)
    ad1 = att_dst1.reshape(D)[pm].reshape(1, D)
    as2 = att_src2.reshape(D)[pm].reshape(1, D)
    ad2 = att_dst2.reshape(D)[pm].reshape(1, D)
    b1v = b1[pm].reshape(1, D)
    b2v = b2[pm].reshape(1, D)
    lbv = lin_b.reshape(1, OUT)
    W1p = W1[:, pm]
    W2p = W2[pm][:, pm]
    lin_Wp = lin_W[pm]

    xw1, asrc1, adst1 = _tc_prep(xp, W1p, as1, ad1)
    packed1 = jnp.concatenate([xw1, asrc1, adst1], axis=1)
    adst1t = jnp.concatenate([adst1, jnp.zeros((NP, ADW - H), f32)], axis=1)
    acc1 = _sc_edge(packed1, adst1t, srcp, dstp)

    xw2, asrc2, adst2 = _tc_mid(acc1, b1v, W2p, as2, ad2)
    packed2 = jnp.concatenate([xw2, asrc2, adst2], axis=1)
    adst2t = jnp.concatenate([adst2, jnp.zeros((NP, ADW - H), f32)], axis=1)
    acc2 = _sc_edge(packed2, adst2t, srcp, dstp)

    return _tc_final(acc2, b2v, batch2d, lin_Wp, lbv)


# speedup vs baseline: 1.6972x; 1.4166x over previous
import functools

import jax
import jax.numpy as jnp
import numpy as np
from jax import lax
from jax.experimental import pallas as pl
from jax.experimental.pallas import tpu as pltpu
from jax.experimental.pallas import tpu_sc as plsc

N = 10000
E = 320000
D = 128
H = 8
C = 16
G = 128
OUT = 16

NC = 2
NS = 16
NW = NC * NS
NP = 10240
EP = 327680
EPW = EP // NW
K = 16
NCHUNK = EPW // K
NBUF = 5
NCH0 = 800
NCH1 = 2 * NCHUNK - NCH0
ROW = 144
RB = 160
ADB = 32
RPT = NP // NS
BLK = 512
NBLK = NP // BLK


_PERM = np.array([(j % 8) * 16 + 2 * (j // 16) + ((j % 16) // 8)
                  for j in range(D)], dtype=np.int32)
_INTL = np.array([32 * (c // 32) + ((c % 32) // 2) + 16 * (c % 2)
                  for c in range(D)], dtype=np.int32)
_PERMB = _PERM[_INTL]


def _head_sel_b(dtype):
    cc = lax.broadcasted_iota(jnp.int32, (D, H), 0)
    hh = lax.broadcasted_iota(jnp.int32, (D, H), 1)
    return jnp.where(((cc >> 1) & 7) == hh, 1.0, 0.0).astype(dtype)


def _tail_sel(off):
    vv = lax.broadcasted_iota(jnp.int32, (H, ADB), 0)
    kk = lax.broadcasted_iota(jnp.int32, (H, ADB), 1)
    return jnp.where(kk == 2 * vv + off, 1.0, 0.0)


def _emit_tables(xw, asv, adv, pk_ref, ad_ref):
    sel = _head_sel_b(jnp.float32)
    asrc = jnp.dot(xw * asv, sel, preferred_element_type=jnp.float32)
    adst = jnp.dot(xw * adv, sel, preferred_element_type=jnp.float32)
    tail = (jnp.dot(asrc, _tail_sel(0), preferred_element_type=jnp.float32)
            + jnp.dot(adst, _tail_sel(16), preferred_element_type=jnp.float32))
    pk_ref[...] = jnp.concatenate([xw, tail], axis=1).astype(jnp.bfloat16)
    ad_ref[...] = jnp.dot(adst, _tail_sel(0),
                          preferred_element_type=jnp.float32).astype(jnp.bfloat16)


def _tc_prep_body(x_ref, w_ref, asv_ref, adv_ref, pk_ref, ad_ref):
    xw = jnp.dot(x_ref[...], w_ref[...], preferred_element_type=jnp.float32)
    _emit_tables(xw, asv_ref[...], adv_ref[...], pk_ref, ad_ref)


def _tc_prep(xp, W, asv, adv):
    return pl.pallas_call(
        _tc_prep_body,
        grid=(NBLK,),
        in_specs=[
            pl.BlockSpec((BLK, D), lambda i: (i, 0)),
            pl.BlockSpec((D, D), lambda i: (0, 0)),
            pl.BlockSpec((1, D), lambda i: (0, 0)),
            pl.BlockSpec((1, D), lambda i: (0, 0)),
        ],
        out_specs=[
            pl.BlockSpec((BLK, RB), lambda i: (i, 0)),
            pl.BlockSpec((BLK, ADB), lambda i: (i, 0)),
        ],
        out_shape=[
            jax.ShapeDtypeStruct((NP, RB), jnp.bfloat16),
            jax.ShapeDtypeStruct((NP, ADB), jnp.bfloat16),
        ],
    )(xp, W, asv, adv)


def _sc_edge_body(packed_hbm, adst_hbm, src_hbm, dst_hbm, out_hbm,
                  srci_v, dsti_v, rows_v, adst_v, sbuf_v, acc_sh,
                  sems_i, sems_g, sems_s):
    c = lax.axis_index("c")
    s = lax.axis_index("s")
    nch = jnp.where(c == 0, NCH0, NCH1)
    base = jnp.where(c == 0, s * NCH0, NS * NCH0 + s * NCH1)

    def zero_row(r, _):
        for j in range(ROW // 16):
            sbuf_v[0, r, pl.ds(j * 16, 16)] = jnp.zeros((16,), jnp.float32)
        return 0
    lax.fori_loop(0, K, zero_row, 0)
    for kk in range(RPT // K):
        pltpu.make_async_copy(sbuf_v.at[0],
                              acc_sh.at[pl.ds(s * RPT + kk * K, K)],
                              sems_g.at[0]).start()
    for kk in range(RPT // K):
        pltpu.make_async_copy(sbuf_v.at[0],
                              acc_sh.at[pl.ds(s * RPT + kk * K, K)],
                              sems_g.at[0]).wait()
    plsc.subcore_barrier()

    lane = lax.iota(jnp.int32, 16)
    idx8 = lane & 7
    zvec = jnp.zeros((16,), jnp.int32)

    def start_i(ci, b):
        pltpu.make_async_copy(src_hbm.at[base + ci], srci_v.at[b],
                              sems_i.at[b]).start()
        pltpu.make_async_copy(dst_hbm.at[base + ci], dsti_v.at[b],
                              sems_i.at[b]).start()

    def wait_i(b):
        pltpu.make_async_copy(src_hbm.at[base], srci_v.at[b],
                              sems_i.at[b]).wait()
        pltpu.make_async_copy(dst_hbm.at[base], dsti_v.at[b],
                              sems_i.at[b]).wait()

    def start_g(b):
        pltpu.make_async_copy(packed_hbm.at[srci_v.at[b]], rows_v.at[b],
                              sems_g.at[b]).start()
        pltpu.make_async_copy(adst_hbm.at[dsti_v.at[b]], adst_v.at[b],
                              sems_g.at[b]).start()

    def wait_g(b):
        pltpu.make_async_copy(packed_hbm.at[srci_v.at[b]], rows_v.at[b],
                              sems_g.at[b]).wait()
        pltpu.make_async_copy(adst_hbm.at[dsti_v.at[b]], adst_v.at[b],
                              sems_g.at[b]).wait()

    def start_s(b, dvec):
        pltpu.make_async_copy(sbuf_v.at[b], acc_sh.at[dvec],
                              sems_s.at[b]).start(add=True)

    def wait_s(b):
        pltpu.make_async_copy(sbuf_v.at[b], acc_sh.at[zvec],
                              sems_s.at[b]).wait()

    def compute(b):
        for i in range(K):
            av, _ = plsc.unpack(rows_v[b, i, pl.ds(D, ADB)],
                                format=plsc.PackFormat.INTERLEAVED)
            dv, _ = plsc.unpack(adst_v[b, i, :],
                                format=plsc.PackFormat.INTERLEAVED)
            a = av + dv
            a = jnp.where(a < 0, a * 0.2, a)
            sv = jnp.exp(a)
            sv = jnp.where(lane < H, sv, 0.0)
            sbuf_v[b, i, pl.ds(D, 16)] = sv
            sp = sv.at[idx8].get(mode="promise_in_bounds")
            for g in range(4):
                xa, xb = plsc.unpack(rows_v[b, i, pl.ds(32 * g, 32)],
                                     format=plsc.PackFormat.INTERLEAVED)
                sbuf_v[b, i, pl.ds(32 * g, 16)] = xa * sp
                sbuf_v[b, i, pl.ds(32 * g + 16, 16)] = xb * sp
        return dsti_v[b, :]

    for b in range(4):
        start_i(b, b)
    wait_i(0)
    start_g(0)
    wait_i(1)
    start_g(1)

    def ring_body(p, _):
        for j in range(NBUF):
            ci = NBUF * p + j
            b2 = (j + 2) % NBUF
            b4 = (j + 4) % NBUF
            wait_g(j)

            @pl.when(ci + 2 < nch)
            def _():
                wait_i(b2)
                start_g(b2)

            @pl.when(ci + 4 < nch)
            def _():
                start_i(ci + 4, b4)

            @pl.when(ci >= NBUF)
            def _():
                wait_s(j)
            dvec = compute(j)
            start_s(j, dvec)
        return 0
    lax.fori_loop(0, nch // NBUF, ring_body, 0)
    for j in range(NBUF):
        wait_s(j)

    plsc.subcore_barrier()
    pltpu.sync_copy(acc_sh.at[pl.ds(s * RPT, RPT)],
                    out_hbm.at[c, pl.ds(s * RPT, RPT)])


def _sc_edge(packed, adst, srcp, dstp):
    return pl.kernel(
        _sc_edge_body,
        out_type=jax.ShapeDtypeStruct((NC, NP, ROW), jnp.float32),
        mesh=plsc.VectorSubcoreMesh(core_axis_name="c", subcore_axis_name="s",
                                    num_cores=NC, num_subcores=NS),
        compiler_params=pltpu.CompilerParams(use_tc_tiling_on_sc=False,
                                             needs_layout_passes=False),
        scratch_types=[
            pltpu.VMEM((NBUF, K), jnp.int32),
            pltpu.VMEM((NBUF, K), jnp.int32),
            pltpu.VMEM((NBUF, K, RB), jnp.bfloat16),
            pltpu.VMEM((NBUF, K, ADB), jnp.bfloat16),
            pltpu.VMEM((NBUF, K, ROW), jnp.float32),
            pltpu.VMEM_SHARED((NP, ROW), jnp.float32),
            pltpu.SemaphoreType.DMA((NBUF,)),
            pltpu.SemaphoreType.DMA((NBUF,)),
            pltpu.SemaphoreType.DMA((NBUF,)),
        ],
    )(packed, adst, srcp, dstp)


def _denom_sel():
    hh = lax.broadcasted_iota(jnp.int32, (H, D), 0)
    jj = lax.broadcasted_iota(jnp.int32, (H, D), 1)
    return jnp.where((jj & 7) == hh, 1.0, 0.0)


def _combine_norm(acc_ref, b_ref, blk_idx):
    a = acc_ref[0] + acc_ref[1]
    msg = a[:, :D]
    dn = a[:, D:D + H]
    d128 = jnp.dot(dn, _denom_sel(), preferred_element_type=jnp.float32)
    hv = msg / (d128 + 1e-16) + b_ref[...]
    hv = jnp.where(hv > 0, hv, jnp.exp(hv) - 1.0)
    rows = blk_idx * BLK + lax.broadcasted_iota(jnp.int32, (BLK, 1), 0)
    return jnp.where(rows < N, hv, 0.0)


def _tc_mid_body(acc_ref, b_ref, w_ref, asv_ref, adv_ref, pk_ref, ad_ref):
    i = pl.program_id(0)
    h1 = _combine_norm(acc_ref, b_ref, i)
    xw = jnp.dot(h1, w_ref[...], preferred_element_type=jnp.float32)
    _emit_tables(xw, asv_ref[...], adv_ref[...], pk_ref, ad_ref)


def _tc_mid(acc, bv, W, asv, adv):
    return pl.pallas_call(
        _tc_mid_body,
        grid=(NBLK,),
        in_specs=[
            pl.BlockSpec((NC, BLK, ROW), lambda i: (0, i, 0)),
            pl.BlockSpec((1, D), lambda i: (0, 0)),
            pl.BlockSpec((D, D), lambda i: (0, 0)),
            pl.BlockSpec((1, D), lambda i: (0, 0)),
            pl.BlockSpec((1, D), lambda i: (0, 0)),
        ],
        out_specs=[
            pl.BlockSpec((BLK, RB), lambda i: (i, 0)),
            pl.BlockSpec((BLK, ADB), lambda i: (i, 0)),
        ],
        out_shape=[
            jax.ShapeDtypeStruct((NP, RB), jnp.bfloat16),
            jax.ShapeDtypeStruct((NP, ADB), jnp.bfloat16),
        ],
    )(acc, bv, W, asv, adv)


def _tc_final_body(acc_ref, b_ref, batch_ref, lw_ref, lb_ref, out_ref, pool_ref):
    i = pl.program_id(0)

    @pl.when(i == 0)
    def _():
        pool_ref[...] = jnp.zeros((G, ROW), jnp.float32)

    h2 = _combine_norm(acc_ref, b_ref, i)
    h2e = jnp.concatenate([h2, jnp.ones((BLK, ROW - D), jnp.float32)], axis=1)
    bv = batch_ref[0]
    gg = lax.broadcasted_iota(jnp.int32, (G, BLK), 0).astype(jnp.float32)
    p = jnp.where(gg == bv, 1.0, 0.0)
    pool_ref[...] += jnp.dot(p, h2e, preferred_element_type=jnp.float32)

    @pl.when(i == NBLK - 1)
    def _():
        sums = pool_ref[:, :D]
        counts = pool_ref[:, D:D + 1]
        pooled = sums / jnp.maximum(counts, 1.0)
        out_ref[...] = jnp.dot(pooled, lw_ref[...],
                               preferred_element_type=jnp.float32) + lb_ref[...]


def _tc_final(acc, bv, batch2d, lw, lb):
    return pl.pallas_call(
        _tc_final_body,
        grid=(NBLK,),
        in_specs=[
            pl.BlockSpec((NC, BLK, ROW), lambda i: (0, i, 0)),
            pl.BlockSpec((1, D), lambda i: (0, 0)),
            pl.BlockSpec((1, 1, BLK), lambda i: (i, 0, 0)),
            pl.BlockSpec((D, OUT), lambda i: (0, 0)),
            pl.BlockSpec((1, OUT), lambda i: (0, 0)),
        ],
        out_specs=pl.BlockSpec((G, OUT), lambda i: (0, 0)),
        out_shape=jax.ShapeDtypeStruct((G, OUT), jnp.float32),
        scratch_shapes=[pltpu.VMEM((G, ROW), jnp.float32)],
    )(acc, bv, batch2d, lw, lb)


def kernel(x, edge_index, batch, W1, att_src1, att_dst1, b1,
           W2, att_src2, att_dst2, b2, lin_W, lin_b):
    f32 = jnp.float32
    xp = jnp.pad(x, ((0, NP - N), (0, 0)))
    srcp = jnp.concatenate([edge_index[0], jnp.full((EP - E,), N, jnp.int32)]).reshape(EP // K, K)
    dstp = jnp.concatenate([edge_index[1], jnp.full((EP - E,), N, jnp.int32)]).reshape(EP // K, K)
    batch2d = jnp.pad(batch, (0, NP - N), constant_values=G).astype(f32).reshape(NBLK, 1, BLK)

    pm = jnp.asarray(_PERM)
    pmb = jnp.asarray(_PERMB)
    as1 = att_src1.reshape(D)[pmb].reshape(1, D)
    ad1 = att_dst1.reshape(D)[pmb].reshape(1, D)
    as2 = att_src2.reshape(D)[pmb].reshape(1, D)
    ad2 = att_dst2.reshape(D)[pmb].reshape(1, D)
    b1v = b1[pm].reshape(1, D)
    b2v = b2[pm].reshape(1, D)
    lbv = lin_b.reshape(1, OUT)
    W1b = W1[:, pmb]
    W2b = W2[pm][:, pmb]
    lin_Wp = lin_W[pm]

    packed1, adst1 = _tc_prep(xp, W1b, as1, ad1)
    acc1 = _sc_edge(packed1, adst1, srcp, dstp)
    packed2, adst2 = _tc_mid(acc1, b1v, W2b, as2, ad2)
    acc2 = _sc_edge(packed2, adst2, srcp, dstp)
    return _tc_final(acc2, b2v, batch2d, lin_Wp, lbv)
